# baseline (speedup 1.0000x reference)
from functools import partial

import numpy as np
import jax
import jax.numpy as jnp
from jax.experimental import pallas as pl
from jax.experimental.pallas import tpu as pltpu
from jax.experimental.pallas import tpu_sc as plsc

_D_MODEL = 1024
_VALUE_RANGE = 4
_NUM_TRIPLES = 969
_TBL_ROWS = 976


def _pos_code_sum(k, d_model):
    positions = np.arange(k, dtype=np.float64)[:, None]
    i = np.arange(0, d_model, 2, dtype=np.float64)
    omega = 1.0 / (10000.0 ** (i / d_model))
    angles = positions * omega[None, :]
    codes = np.zeros((k, d_model), np.float64)
    codes[:, 0::2] = np.sin(angles)
    codes[:, 1::2] = np.cos(angles)
    return codes.sum(axis=0).astype(np.float32)


def _triple_columns(kmax):
    c1s, c2s, c3s = [], [], []
    for c3 in range(kmax + 1):
        for c2 in range(kmax + 1 - c3):
            for c1 in range(kmax + 1 - c3 - c2):
                c1s.append(c1)
                c2s.append(c2)
                c3s.append(c3)
    pad = _TBL_ROWS - len(c1s)
    for _ in range(pad):
        c1s.append(0)
        c2s.append(0)
        c3s.append(0)
    cols = lambda x: np.asarray(x, np.float32)[:, None]
    return cols(c1s), cols(c2s), cols(c3s)


def _stage_a(ch_ref, ve_ref, ps_ref, b_ref, mix_ref, w_ref,
             t1_ref, t2_ref, t3_ref, tbl_ref, idx_ref, *, k):
    d = ve_ref.shape[1]
    a = jnp.concatenate(
        [ve_ref[...], ps_ref[...], jnp.zeros((3, d), jnp.float32)], axis=0)
    raw = jax.lax.dot_general(a, w_ref[...], (((1,), (1,)), ((), ())),
                              preferred_element_type=jnp.float32)
    mix = mix_ref[0]
    base = mix * (raw[4:5] + b_ref[...] + float(k) * raw[0:1])
    deltas = mix * (raw[1:4] - raw[0:1])

    slab = 8
    bb = jnp.broadcast_to(base, (slab, d))
    b1 = jnp.broadcast_to(deltas[0:1], (slab, d))
    b2 = jnp.broadcast_to(deltas[1:2], (slab, d))
    b3 = jnp.broadcast_to(deltas[2:3], (slab, d))
    for i in range(_TBL_ROWS // slab):
        sl = pl.ds(i * slab, slab)
        tbl_ref[sl, :] = (bb + t1_ref[sl, :] * b1 + t2_ref[sl, :] * b2
                          + t3_ref[sl, :] * b3)

    ch = jnp.clip(ch_ref[...], 0, _VALUE_RANGE - 1)
    c1 = jnp.sum((ch == 1).astype(jnp.float32), axis=1, keepdims=True)
    c2 = jnp.sum((ch == 2).astype(jnp.float32), axis=1, keepdims=True)
    c3 = jnp.sum((ch == 3).astype(jnp.float32), axis=1, keepdims=True)
    m = c3
    fm = (153.0 * m - 8.75 * (m * (m - 1.0))
          + ((m - 1.0) * m * (2.0 * m - 1.0)) * (1.0 / 12.0))
    rank = fm + c2 * (17.0 - c3) - 0.5 * (c2 * (c2 - 1.0)) + c1
    seg = jax.lax.broadcasted_iota(jnp.int32, (rank.shape[0], 8), 1)
    idx_ref[...] = jnp.round(rank).astype(jnp.int32) * 8 + seg


def _sc_gather(table128, idx2d, n_idx):
    mesh = plsc.VectorSubcoreMesh(core_axis_name="c", subcore_axis_name="s")
    window = 128

    @partial(pl.kernel,
             out_type=jax.ShapeDtypeStruct((n_idx, 128), jnp.float32),
             mesh=mesh)
    def gather_kernel(tbl_hbm, i_hbm, o_hbm):
        def body(i_vmem, o_vmem):
            pltpu.sync_copy(tbl_hbm.at[i_vmem.at[0]], o_vmem)

        pltpu.emit_pipeline(
            body,
            grid=(n_idx // window,),
            in_specs=[pl.BlockSpec((1, window), lambda i: (0, i))],
            out_specs=[pl.BlockSpec((window, 128), lambda i: (i, 0))],
            core_axis_name=("c", "s"),
            dimension_semantics=(pltpu.PARALLEL,),
        )(i_hbm, o_hbm)

    return gather_kernel(table128, idx2d)


def kernel(channels, value_emb, read_W, read_b, mix):
    B, L, K = channels.shape
    N = B * L
    ch2d = channels.reshape(N, K)
    pos_sum = jnp.asarray(_pos_code_sum(K, _D_MODEL))[None, :]
    b2d = read_b[None, :]
    mix1 = jnp.asarray(mix, jnp.float32).reshape(1)
    t1, t2, t3 = (jnp.asarray(x) for x in _triple_columns(K))

    whole = lambda: (0, 0)
    table, idx = pl.pallas_call(
        partial(_stage_a, k=K),
        in_specs=[pl.BlockSpec((N, K), whole),
                  pl.BlockSpec((_VALUE_RANGE, _D_MODEL), whole),
                  pl.BlockSpec((1, _D_MODEL), whole),
                  pl.BlockSpec((1, _D_MODEL), whole),
                  pl.BlockSpec(memory_space=pltpu.SMEM),
                  pl.BlockSpec((_D_MODEL, _D_MODEL), whole),
                  pl.BlockSpec((_TBL_ROWS, 1), whole),
                  pl.BlockSpec((_TBL_ROWS, 1), whole),
                  pl.BlockSpec((_TBL_ROWS, 1), whole)],
        out_specs=[pl.BlockSpec((_TBL_ROWS, _D_MODEL), whole),
                   pl.BlockSpec((N, 8), whole)],
        out_shape=[jax.ShapeDtypeStruct((_TBL_ROWS, _D_MODEL), jnp.float32),
                   jax.ShapeDtypeStruct((N, 8), jnp.int32)],
    )(ch2d, value_emb, pos_sum, b2d, mix1, read_W, t1, t2, t3)

    segs = _D_MODEL // 128
    out128 = _sc_gather(table.reshape(_TBL_ROWS * segs, 128),
                        idx.reshape(1, N * segs), N * segs)
    return out128.reshape(B, L, _D_MODEL)


# speedup vs baseline: 4.7102x; 4.7102x over previous
from functools import partial

import numpy as np
import jax
import jax.numpy as jnp
from jax.experimental import pallas as pl
from jax.experimental.pallas import tpu as pltpu

_D_MODEL = 1024
_VALUE_RANGE = 4
_ROW_TILE = 2048
_COL_SPLIT = 2


def _pos_code_sum(k, d_model):
    positions = np.arange(k, dtype=np.float64)[:, None]
    i = np.arange(0, d_model, 2, dtype=np.float64)
    omega = 1.0 / (10000.0 ** (i / d_model))
    angles = positions * omega[None, :]
    codes = np.zeros((k, d_model), np.float64)
    codes[:, 0::2] = np.sin(angles)
    codes[:, 1::2] = np.cos(angles)
    return codes.sum(axis=0).astype(np.float32)


def _body(ch_ref, ve_ref, ps_ref, b_ref, mix_ref, w_ref, o_ref, p_ref,
          *, tile, k, slab=8):
    dc = o_ref.shape[1]

    @pl.when(pl.program_id(1) == 0)
    def _prep():
        a = jnp.concatenate(
            [ve_ref[...], ps_ref[...],
             jnp.zeros((3, ve_ref.shape[1]), jnp.float32)], axis=0)
        raw = jax.lax.dot_general(a, w_ref[...], (((1,), (1,)), ((), ())),
                                  preferred_element_type=jnp.float32)
        mix = mix_ref[0]
        base = mix * (raw[4:5] + b_ref[...] + float(k) * raw[0:1])
        deltas = mix * (raw[1:4] - raw[0:1])
        p_ref[...] = jnp.concatenate(
            [base, deltas, jnp.zeros((4, dc), jnp.float32)], axis=0)

    bb = jnp.broadcast_to(p_ref[0:1, :], (slab, dc))
    b1 = jnp.broadcast_to(p_ref[1:2, :], (slab, dc))
    b2 = jnp.broadcast_to(p_ref[2:3, :], (slab, dc))
    b3 = jnp.broadcast_to(p_ref[3:4, :], (slab, dc))

    for i in range(tile // slab):
        r = i * slab
        ch = jnp.clip(ch_ref[pl.ds(r, slab), :], 0, _VALUE_RANGE - 1)
        c1 = jnp.sum((ch == 1).astype(jnp.float32), axis=1, keepdims=True)
        c2 = jnp.sum((ch == 2).astype(jnp.float32), axis=1, keepdims=True)
        c3 = jnp.sum((ch == 3).astype(jnp.float32), axis=1, keepdims=True)
        o_ref[pl.ds(r, slab), :] = bb + c1 * b1 + c2 * b2 + c3 * b3


def kernel(channels, value_emb, read_W, read_b, mix):
    B, L, K = channels.shape
    N = B * L
    ch2d = channels.reshape(N, K)
    pos_sum = jnp.asarray(_pos_code_sum(K, _D_MODEL))[None, :]
    b2d = read_b[None, :]
    mix1 = jnp.asarray(mix, jnp.float32).reshape(1)

    T = _ROW_TILE
    C = _D_MODEL // _COL_SPLIT
    out2d = pl.pallas_call(
        partial(_body, tile=T, k=K),
        grid=(_COL_SPLIT, N // T),
        in_specs=[pl.BlockSpec((T, K), lambda j, i: (i, 0)),
                  pl.BlockSpec((_VALUE_RANGE, _D_MODEL), lambda j, i: (0, 0)),
                  pl.BlockSpec((1, _D_MODEL), lambda j, i: (0, 0)),
                  pl.BlockSpec((1, C), lambda j, i: (0, j)),
                  pl.BlockSpec(memory_space=pltpu.SMEM),
                  pl.BlockSpec((C, _D_MODEL), lambda j, i: (j, 0))],
        out_specs=pl.BlockSpec((T, C), lambda j, i: (i, j)),
        out_shape=jax.ShapeDtypeStruct((N, _D_MODEL), jnp.float32),
        scratch_shapes=[pltpu.VMEM((8, C), jnp.float32)],
        compiler_params=pltpu.CompilerParams(
            dimension_semantics=("arbitrary", "arbitrary")),
    )(ch2d, value_emb, pos_sum, b2d, mix1, read_W)
    return out2d.reshape(B, L, _D_MODEL)


# speedup vs baseline: 5.4341x; 1.1537x over previous
from functools import partial

import numpy as np
import jax
import jax.numpy as jnp
from jax.experimental import pallas as pl
from jax.experimental.pallas import tpu as pltpu

_D_MODEL = 1024
_VALUE_RANGE = 4
_ROW_TILE = 2048


def _pos_code_sum(k, d_model):
    positions = np.arange(k, dtype=np.float64)[:, None]
    i = np.arange(0, d_model, 2, dtype=np.float64)
    omega = 1.0 / (10000.0 ** (i / d_model))
    angles = positions * omega[None, :]
    codes = np.zeros((k, d_model), np.float64)
    codes[:, 0::2] = np.sin(angles)
    codes[:, 1::2] = np.cos(angles)
    return codes.sum(axis=0).astype(np.float32)


def _body(ch_ref, ve_ref, ps_ref, b_ref, mix_ref, w_ref, o_ref, p_ref,
          *, tile, k):
    d = ve_ref.shape[1]

    @pl.when(pl.program_id(0) == 0)
    def _prep():
        a = jnp.concatenate(
            [ve_ref[...], ps_ref[...], jnp.zeros((3, d), jnp.float32)], axis=0)
        raw = jax.lax.dot_general(a, w_ref[...], (((1,), (1,)), ((), ())),
                                  preferred_element_type=jnp.float32)
        mix = mix_ref[0]
        base = mix * (raw[4:5] + b_ref[...] + float(k) * raw[0:1])
        deltas = mix * (raw[1:4] - raw[0:1])
        p_ref[...] = jnp.concatenate(
            [base, deltas, jnp.zeros((4, d), jnp.float32)], axis=0)

    slab = 8
    bb = jnp.broadcast_to(p_ref[0:1, :], (slab, d))
    b1 = jnp.broadcast_to(p_ref[1:2, :], (slab, d))
    b2 = jnp.broadcast_to(p_ref[2:3, :], (slab, d))
    b3 = jnp.broadcast_to(p_ref[3:4, :], (slab, d))
    for i in range(tile // slab):
        r = i * slab
        ch = jnp.clip(ch_ref[pl.ds(r, slab), :], 0, _VALUE_RANGE - 1)
        c1 = jnp.sum((ch == 1).astype(jnp.float32), axis=1, keepdims=True)
        c2 = jnp.sum((ch == 2).astype(jnp.float32), axis=1, keepdims=True)
        c3 = jnp.sum((ch == 3).astype(jnp.float32), axis=1, keepdims=True)
        o_ref[pl.ds(r, slab), :] = bb + c1 * b1 + c2 * b2 + c3 * b3


def kernel(channels, value_emb, read_W, read_b, mix):
    B, L, K = channels.shape
    N = B * L
    ch2d = channels.reshape(N, K)
    pos_sum = jnp.asarray(_pos_code_sum(K, _D_MODEL))[None, :]
    b2d = read_b[None, :]
    mix1 = jnp.asarray(mix, jnp.float32).reshape(1)

    T = _ROW_TILE
    whole = lambda i: (0, 0)
    out2d = pl.pallas_call(
        partial(_body, tile=T, k=K),
        grid=(N // T,),
        in_specs=[pl.BlockSpec((T, K), lambda i: (i, 0)),
                  pl.BlockSpec((_VALUE_RANGE, _D_MODEL), whole),
                  pl.BlockSpec((1, _D_MODEL), whole),
                  pl.BlockSpec((1, _D_MODEL), whole),
                  pl.BlockSpec(memory_space=pltpu.SMEM),
                  pl.BlockSpec((_D_MODEL, _D_MODEL), whole)],
        out_specs=pl.BlockSpec((T, _D_MODEL), lambda i: (i, 0)),
        out_shape=jax.ShapeDtypeStruct((N, _D_MODEL), jnp.float32),
        scratch_shapes=[pltpu.VMEM((8, _D_MODEL), jnp.float32)],
        compiler_params=pltpu.CompilerParams(
            dimension_semantics=("parallel",)),
    )(ch2d, value_emb, pos_sum, b2d, mix1, read_W)
    return out2d.reshape(B, L, _D_MODEL)
